# Initial kernel scaffold; baseline (speedup 1.0000x reference)
import jax
import jax.numpy as jnp
from jax.experimental import pallas as pl


def kernel(x, edge_index, W_in, b_in, att_l0, att_r0, att_l1, att_r1, W_out, b_out):
    raise NotImplementedError("write your pallas kernel here")


# speedup vs baseline: 29.4087x; 29.4087x over previous
import functools

import jax
import jax.numpy as jnp
from jax import lax
from jax.experimental import pallas as pl
from jax.experimental.pallas import tpu as pltpu
from jax.experimental.pallas import tpu_sc as plsc

EPS = 0.1
NC = 2
NS = 16
NW = NC * NS
CHUNK = 128


def _sc_mesh():
    return plsc.VectorSubcoreMesh(
        core_axis_name="c", subcore_axis_name="s", num_cores=NC, num_subcores=NS
    )


def kernel(x, edge_index, W_in, b_in, att_l0, att_r0, att_l1, att_r1, W_out, b_out):
    N, D = x.shape
    H = W_in.shape[1]
    E = edge_index.shape[1]

    NP = -(-N // (NS * CHUNK)) * (NS * CHUNK)
    SUB = NP // NS
    n_chunks = -(-E // (NW * CHUNK))
    E_pad = n_chunks * NW * CHUNK

    src = edge_index[0]
    dst = edge_index[1]
    pad_n = E_pad - E
    pad_idx = (jnp.arange(pad_n, dtype=jnp.int32) % N).astype(jnp.int32)
    srcp = jnp.concatenate([src, pad_idx]).reshape(NW, n_chunks, CHUNK)
    dstp = jnp.concatenate([dst, pad_idx]).reshape(NW, n_chunks, CHUNK)
    val = jnp.concatenate(
        [jnp.ones((E,), jnp.float32), jnp.zeros((pad_n,), jnp.float32)]
    ).reshape(NW, n_chunks, CHUNK)
    xp = jnp.pad(x, ((0, NP - N), (0, 0)))

    @functools.partial(
        pl.kernel,
        out_type=jax.ShapeDtypeStruct((NC, NP), jnp.float32),
        mesh=_sc_mesh(),
        compiler_params=pltpu.CompilerParams(needs_layout_passes=False),
        scratch_types=[
            pltpu.VMEM((n_chunks, CHUNK), jnp.int32),
            pltpu.VMEM((n_chunks, CHUNK), jnp.float32),
            pltpu.VMEM((SUB,), jnp.float32),
            pltpu.VMEM_SHARED((NP,), jnp.float32),
        ],
    )
    def deg_kernel(dst_hbm, val_hbm, out_hbm, dst_v, val_v, zbuf_v, hist_s):
        cc = lax.axis_index("c")
        ss = lax.axis_index("s")
        wid = ss * NC + cc
        pltpu.sync_copy(dst_hbm.at[wid], dst_v)
        pltpu.sync_copy(val_hbm.at[wid], val_v)

        def zb(i, _):
            zbuf_v[pl.ds(i * 16, 16)] = jnp.zeros((16,), jnp.float32)
            return 0

        lax.fori_loop(0, SUB // 16, zb, 0)
        pltpu.sync_copy(zbuf_v, hist_s.at[pl.ds(ss * SUB, SUB)])
        plsc.subcore_barrier()

        def body(ci, _):
            pltpu.sync_copy(val_v.at[ci], hist_s.at[dst_v.at[ci]], add=True)
            return 0

        lax.fori_loop(0, n_chunks, body, 0)
        plsc.subcore_barrier()
        pltpu.sync_copy(
            hist_s.at[pl.ds(ss * SUB, SUB)], out_hbm.at[cc, pl.ds(ss * SUB, SUB)]
        )

    degp = deg_kernel(dstp, val)

    def tc_a_body(x_ref, w_ref, b_ref, atl_ref, atr_ref, degp_ref,
                  h_ref, al_ref, ar_ref, dinv_ref):
        h = jnp.maximum(
            jnp.dot(x_ref[...], w_ref[...], preferred_element_type=jnp.float32)
            + b_ref[...][None, :],
            0.0,
        )
        h_ref[...] = h
        al_ref[...] = jnp.sum(h * atl_ref[...][None, :], axis=1)
        ar_ref[...] = jnp.sum(h * atr_ref[...][None, :], axis=1)
        deg = degp_ref[0, :] + degp_ref[1, :] + 1.0
        dinv_ref[...] = lax.rsqrt(deg)

    h0, al0, ar0, dinv = pl.pallas_call(
        tc_a_body,
        out_shape=[
            jax.ShapeDtypeStruct((NP, H), jnp.float32),
            jax.ShapeDtypeStruct((NP,), jnp.float32),
            jax.ShapeDtypeStruct((NP,), jnp.float32),
            jax.ShapeDtypeStruct((NP,), jnp.float32),
        ],
    )(xp, W_in, b_in, att_l0, att_r0, degp)

    @functools.partial(
        pl.kernel,
        out_type=jax.ShapeDtypeStruct((NC, NP, H), jnp.float32),
        mesh=_sc_mesh(),
        compiler_params=pltpu.CompilerParams(needs_layout_passes=False),
        scratch_types=[
            pltpu.VMEM((n_chunks, CHUNK), jnp.int32),
            pltpu.VMEM((n_chunks, CHUNK), jnp.int32),
            pltpu.VMEM((CHUNK,), jnp.float32),
            pltpu.VMEM((CHUNK,), jnp.float32),
            pltpu.VMEM((CHUNK,), jnp.float32),
            pltpu.VMEM((CHUNK,), jnp.float32),
            pltpu.VMEM((CHUNK,), jnp.float32),
            pltpu.VMEM((CHUNK, 128), jnp.float32),
            pltpu.VMEM_SHARED((NP,), jnp.float32),
            pltpu.VMEM_SHARED((NP,), jnp.float32),
            pltpu.VMEM_SHARED((NP,), jnp.float32),
            pltpu.VMEM_SHARED((NP, 128), jnp.float32),
            pltpu.SemaphoreType.DMA,
        ],
    )
    def edge_kernel(h_hbm, al_hbm, ar_hbm, dinv_hbm, src_hbm, dst_hbm,
                    out_hbm, src_v, dst_v, als_v, ard_v, dis_v, did_v, coef_v,
                    rows_v, al_sh, ar_sh, dinv_sh, agg_s, sem):
        cc = lax.axis_index("c")
        ss = lax.axis_index("s")
        wid = ss * NC + cc
        pltpu.sync_copy(src_hbm.at[wid], src_v)
        pltpu.sync_copy(dst_hbm.at[wid], dst_v)

        @pl.when(ss == 0)
        def _():
            pltpu.sync_copy(al_hbm, al_sh)

        @pl.when(ss == 1)
        def _():
            pltpu.sync_copy(ar_hbm, ar_sh)

        @pl.when(ss == 2)
        def _():
            pltpu.sync_copy(dinv_hbm, dinv_sh)

        def zrow(i, _):
            for j in range(8):
                rows_v[i, pl.ds(j * 16, 16)] = jnp.zeros((16,), jnp.float32)
            return 0

        lax.fori_loop(0, CHUNK, zrow, 0)
        for k in range(SUB // CHUNK):
            pltpu.sync_copy(rows_v, agg_s.at[pl.ds(ss * SUB + k * CHUNK, CHUNK)])
        plsc.subcore_barrier()

        def chunk_body(ci, _):
            pltpu.async_copy(h_hbm.at[src_v.at[ci]], rows_v, sem).wait()
            pltpu.sync_copy(al_sh.at[src_v.at[ci]], als_v)
            pltpu.sync_copy(ar_sh.at[dst_v.at[ci]], ard_v)
            pltpu.sync_copy(dinv_sh.at[src_v.at[ci]], dis_v)
            pltpu.sync_copy(dinv_sh.at[dst_v.at[ci]], did_v)
            base_gid = (wid * n_chunks + ci) * CHUNK
            for g in range(8):
                av = als_v[pl.ds(g * 16, 16)]
                rv = ard_v[pl.ds(g * 16, 16)]
                dsv = dis_v[pl.ds(g * 16, 16)]
                ddv = did_v[pl.ds(g * 16, 16)]
                gid = base_gid + g * 16 + lax.iota(jnp.int32, 16)
                z = jnp.clip(av + rv, -15.0, 15.0)
                e = jnp.exp(2.0 * z)
                t = (e - 1.0) / (e + 1.0)
                coef_v[pl.ds(g * 16, 16)] = jnp.where(
                    gid < E, t * dsv * ddv, 0.0
                )

            def srow(g, _):
                cvec = coef_v[pl.ds(g * 16, 16)]
                for r16 in range(16):
                    cs = cvec[r16]
                    row = g * 16 + r16
                    for j in range(8):
                        sl = rows_v[row, pl.ds(j * 16, 16)]
                        rows_v[row, pl.ds(j * 16, 16)] = sl * cs
                return 0

            lax.fori_loop(0, CHUNK // 16, srow, 0)
            pltpu.sync_copy(rows_v, agg_s.at[dst_v.at[ci]], add=True)
            return 0

        lax.fori_loop(0, n_chunks, chunk_body, 0)
        plsc.subcore_barrier()
        pltpu.sync_copy(
            agg_s.at[pl.ds(ss * SUB, SUB)], out_hbm.at[cc, pl.ds(ss * SUB, SUB)]
        )

    def tc_c_body(aggp_ref, hprev_ref, h0_ref, al_ref, ar_ref, dinv_ref,
                  atl_ref, atr_ref, hn_ref, aln_ref, arn_ref):
        agg = aggp_ref[0] + aggp_ref[1]
        dinv_n = dinv_ref[...]
        cself = jnp.tanh(al_ref[...] + ar_ref[...]) * dinv_n * dinv_n
        hn = agg + hprev_ref[...] * cself[:, None] + EPS * h0_ref[...]
        hn_ref[...] = hn
        aln_ref[...] = jnp.sum(hn * atl_ref[...][None, :], axis=1)
        arn_ref[...] = jnp.sum(hn * atr_ref[...][None, :], axis=1)

    def tc_combine(aggp, hprev, al, ar, att_l_next, att_r_next):
        return pl.pallas_call(
            tc_c_body,
            out_shape=[
                jax.ShapeDtypeStruct((NP, H), jnp.float32),
                jax.ShapeDtypeStruct((NP,), jnp.float32),
                jax.ShapeDtypeStruct((NP,), jnp.float32),
            ],
        )(aggp, hprev, h0, al, ar, dinv, att_l_next, att_r_next)

    aggp1 = edge_kernel(h0, al0, ar0, dinv, srcp, dstp)
    h1, al1, ar1 = tc_combine(aggp1, h0, al0, ar0, att_l1, att_r1)
    aggp2 = edge_kernel(h1, al1, ar1, dinv, srcp, dstp)

    def tc_e_body(aggp_ref, hprev_ref, h0_ref, al_ref, ar_ref, dinv_ref,
                  wout_ref, bout_ref, hout_ref, gsum_ref):
        agg = aggp_ref[0] + aggp_ref[1]
        dinv_n = dinv_ref[...]
        cself = jnp.tanh(al_ref[...] + ar_ref[...]) * dinv_n * dinv_n
        h2 = agg + hprev_ref[...] * cself[:, None] + EPS * h0_ref[...]
        rows = lax.broadcasted_iota(jnp.int32, (NP, H), 0)
        h2 = jnp.where(rows < N, h2, 0.0)
        wout = wout_ref[...]
        hout = (
            jnp.dot(h2, wout, preferred_element_type=jnp.float32)
            + bout_ref[...][None, :]
        )
        hout_ref[...] = hout
        colsum = jnp.sum(h2, axis=0)
        gsum_ref[...] = (
            jnp.dot(colsum, wout, preferred_element_type=jnp.float32)
            + float(N) * bout_ref[...]
        )[None, :]

    hout_p, gsum = pl.pallas_call(
        tc_e_body,
        out_shape=[
            jax.ShapeDtypeStruct((NP, H), jnp.float32),
            jax.ShapeDtypeStruct((1, H), jnp.float32),
        ],
    )(aggp2, h1, h0, al1, ar1, dinv, W_out, b_out)

    return gsum, hout_p[:N]


# speedup vs baseline: 43.5820x; 1.4819x over previous
import functools

import jax
import jax.numpy as jnp
from jax import lax
from jax.experimental import pallas as pl
from jax.experimental.pallas import tpu as pltpu
from jax.experimental.pallas import tpu_sc as plsc

EPS = 0.1
NC = 2
NS = 16
NW = NC * NS
CHUNK = 64
NSLOT = 3


def _sc_mesh():
    return plsc.VectorSubcoreMesh(
        core_axis_name="c", subcore_axis_name="s", num_cores=NC, num_subcores=NS
    )


def kernel(x, edge_index, W_in, b_in, att_l0, att_r0, att_l1, att_r1, W_out, b_out):
    N, D = x.shape
    H = W_in.shape[1]
    E = edge_index.shape[1]

    NP = -(-N // (NS * CHUNK)) * (NS * CHUNK)
    SUB = NP // NS
    n_chunks = -(--(-E // (NW * CHUNK)) // NSLOT) * NSLOT
    E_pad = n_chunks * NW * CHUNK

    src = edge_index[0]
    dst = edge_index[1]
    pad_n = E_pad - E
    pad_idx = (jnp.arange(pad_n, dtype=jnp.int32) % N).astype(jnp.int32)
    srcp = jnp.concatenate([src, pad_idx]).reshape(NW, n_chunks, CHUNK)
    dstp = jnp.concatenate([dst, pad_idx]).reshape(NW, n_chunks, CHUNK)
    val = jnp.concatenate(
        [jnp.ones((E,), jnp.float32), jnp.zeros((pad_n,), jnp.float32)]
    ).reshape(NW, n_chunks, CHUNK)
    xp = jnp.pad(x, ((0, NP - N), (0, 0)))

    @functools.partial(
        pl.kernel,
        out_type=jax.ShapeDtypeStruct((NC, NP), jnp.float32),
        mesh=_sc_mesh(),
        compiler_params=pltpu.CompilerParams(needs_layout_passes=False),
        scratch_types=[
            pltpu.VMEM((n_chunks, CHUNK), jnp.int32),
            pltpu.VMEM((n_chunks, CHUNK), jnp.float32),
            pltpu.VMEM((SUB,), jnp.float32),
            pltpu.VMEM_SHARED((NP,), jnp.float32),
        ],
    )
    def deg_kernel(dst_hbm, val_hbm, out_hbm, dst_v, val_v, zbuf_v, hist_s):
        cc = lax.axis_index("c")
        ss = lax.axis_index("s")
        wid = ss * NC + cc
        pltpu.sync_copy(dst_hbm.at[wid], dst_v)
        pltpu.sync_copy(val_hbm.at[wid], val_v)

        def zb(i, _):
            zbuf_v[pl.ds(i * 16, 16)] = jnp.zeros((16,), jnp.float32)
            return 0

        lax.fori_loop(0, SUB // 16, zb, 0)
        pltpu.sync_copy(zbuf_v, hist_s.at[pl.ds(ss * SUB, SUB)])
        plsc.subcore_barrier()

        def body(ci, _):
            pltpu.sync_copy(val_v.at[ci], hist_s.at[dst_v.at[ci]], add=True)
            return 0

        lax.fori_loop(0, n_chunks, body, 0)
        plsc.subcore_barrier()
        pltpu.sync_copy(
            hist_s.at[pl.ds(ss * SUB, SUB)], out_hbm.at[cc, pl.ds(ss * SUB, SUB)]
        )

    degp = deg_kernel(dstp, val)

    def tc_a_body(x_ref, w_ref, b_ref, atl_ref, atr_ref, degp_ref,
                  h_ref, al_ref, ar_ref, dinv_ref):
        h = jnp.maximum(
            jnp.dot(x_ref[...], w_ref[...], preferred_element_type=jnp.float32)
            + b_ref[...][None, :],
            0.0,
        )
        h_ref[...] = h
        al_ref[...] = jnp.sum(h * atl_ref[...][None, :], axis=1)
        ar_ref[...] = jnp.sum(h * atr_ref[...][None, :], axis=1)
        deg = degp_ref[0, :] + degp_ref[1, :] + 1.0
        dinv_ref[...] = lax.rsqrt(deg)

    h0, al0, ar0, dinv = pl.pallas_call(
        tc_a_body,
        out_shape=[
            jax.ShapeDtypeStruct((NP, H), jnp.float32),
            jax.ShapeDtypeStruct((NP,), jnp.float32),
            jax.ShapeDtypeStruct((NP,), jnp.float32),
            jax.ShapeDtypeStruct((NP,), jnp.float32),
        ],
    )(xp, W_in, b_in, att_l0, att_r0, degp)

    @functools.partial(
        pl.kernel,
        out_type=jax.ShapeDtypeStruct((NC, NP, H), jnp.float32),
        mesh=_sc_mesh(),
        compiler_params=pltpu.CompilerParams(needs_layout_passes=False),
        scratch_types=[
            [pltpu.VMEM((CHUNK,), jnp.int32)] * NSLOT,
            [pltpu.VMEM((CHUNK,), jnp.int32)] * NSLOT,
            [pltpu.VMEM((CHUNK,), jnp.float32)] * NSLOT,
            [pltpu.VMEM((CHUNK,), jnp.float32)] * NSLOT,
            [pltpu.VMEM((CHUNK,), jnp.float32)] * NSLOT,
            [pltpu.VMEM((CHUNK,), jnp.float32)] * NSLOT,
            [pltpu.VMEM((CHUNK,), jnp.float32)] * NSLOT,
            [pltpu.VMEM((CHUNK, 128), jnp.float32)] * NSLOT,
            pltpu.VMEM_SHARED((NP,), jnp.float32),
            pltpu.VMEM_SHARED((NP,), jnp.float32),
            pltpu.VMEM_SHARED((NP,), jnp.float32),
            pltpu.VMEM_SHARED((NP, 128), jnp.float32),
            [pltpu.SemaphoreType.DMA] * NSLOT,
            [pltpu.SemaphoreType.DMA] * NSLOT,
            [pltpu.SemaphoreType.DMA] * NSLOT,
            [pltpu.SemaphoreType.DMA] * NSLOT,
        ],
    )
    def edge_kernel(h_hbm, al_hbm, ar_hbm, dinv_hbm, src_hbm, dst_hbm,
                    out_hbm, srcb, dstb, als, ard, dis, did, coef,
                    rows, al_sh, ar_sh, dinv_sh, agg_s,
                    sem_i, sem_r, sem_c, sem_s):
        cc = lax.axis_index("c")
        ss = lax.axis_index("s")
        wid = ss * NC + cc

        @pl.when(ss == 0)
        def _():
            pltpu.sync_copy(al_hbm, al_sh)

        @pl.when(ss == 1)
        def _():
            pltpu.sync_copy(ar_hbm, ar_sh)

        @pl.when(ss == 2)
        def _():
            pltpu.sync_copy(dinv_hbm, dinv_sh)

        def zrow(i, _):
            for j in range(8):
                rows[0][i, pl.ds(j * 16, 16)] = jnp.zeros((16,), jnp.float32)
            return 0

        lax.fori_loop(0, CHUNK, zrow, 0)
        for k in range(SUB // CHUNK):
            pltpu.sync_copy(rows[0], agg_s.at[pl.ds(ss * SUB + k * CHUNK, CHUNK)])
        plsc.subcore_barrier()

        def idx_dma(c, k):
            pltpu.async_copy(src_hbm.at[wid, c], srcb[k], sem_i[k])
            pltpu.async_copy(dst_hbm.at[wid, c], dstb[k], sem_i[k])

        def wait_idx(k):
            pltpu.make_async_copy(src_hbm.at[wid, 0], srcb[k], sem_i[k]).wait()
            pltpu.make_async_copy(dst_hbm.at[wid, 0], dstb[k], sem_i[k]).wait()

        def issue_work(k):
            pltpu.async_copy(h_hbm.at[srcb[k]], rows[k], sem_r[k])
            pltpu.async_copy(al_sh.at[srcb[k]], als[k], sem_c[k])
            pltpu.async_copy(ar_sh.at[dstb[k]], ard[k], sem_c[k])
            pltpu.async_copy(dinv_sh.at[srcb[k]], dis[k], sem_c[k])
            pltpu.async_copy(dinv_sh.at[dstb[k]], did[k], sem_c[k])

        def wait_sct(k):
            pltpu.make_async_copy(rows[k], agg_s.at[dstb[k]], sem_s[k]).wait()

        def process(c, k):
            k1 = (k + 1) % NSLOT
            k2 = (k + 2) % NSLOT

            @pl.when(c + 1 < n_chunks)
            def _():
                wait_idx(k1)
                issue_work(k1)

            for _ in range(4):
                pltpu.make_async_copy(
                    al_sh.at[srcb[k]], als[k], sem_c[k]).wait()
            base_gid = (wid * n_chunks + c) * CHUNK
            for g in range(CHUNK // 16):
                av = als[k][pl.ds(g * 16, 16)]
                rv = ard[k][pl.ds(g * 16, 16)]
                dsv = dis[k][pl.ds(g * 16, 16)]
                ddv = did[k][pl.ds(g * 16, 16)]
                gid = base_gid + g * 16 + lax.iota(jnp.int32, 16)
                z = jnp.clip(av + rv, -15.0, 15.0)
                e = jnp.exp(2.0 * z)
                t = (e - 1.0) / (e + 1.0)
                coef[k][pl.ds(g * 16, 16)] = jnp.where(
                    gid < E, t * dsv * ddv, 0.0
                )

            pltpu.make_async_copy(h_hbm.at[srcb[k]], rows[k], sem_r[k]).wait()

            def srow(g, _):
                cvec = coef[k][pl.ds(g * 16, 16)]
                for r16 in range(16):
                    cs = cvec[r16]
                    row = g * 16 + r16
                    for j in range(8):
                        sl = rows[k][row, pl.ds(j * 16, 16)]
                        rows[k][row, pl.ds(j * 16, 16)] = sl * cs
                return 0

            lax.fori_loop(0, CHUNK // 16, srow, 0)
            pltpu.async_copy(rows[k], agg_s.at[dstb[k]], sem_s[k], add=True)

            @pl.when(c >= 1)
            def _():
                wait_sct(k2)

            @pl.when(c + 2 < n_chunks)
            def _():
                idx_dma(c + 2, k2)

        idx_dma(0, 0)
        idx_dma(1, 1)
        wait_idx(0)
        issue_work(0)

        def pipe_body(t, _):
            for k in range(NSLOT):
                process(NSLOT * t + k, k)
            return 0

        lax.fori_loop(0, n_chunks // NSLOT, pipe_body, 0)
        wait_sct((n_chunks - 1) % NSLOT)
        plsc.subcore_barrier()
        pltpu.sync_copy(
            agg_s.at[pl.ds(ss * SUB, SUB)], out_hbm.at[cc, pl.ds(ss * SUB, SUB)]
        )

    def tc_c_body(aggp_ref, hprev_ref, h0_ref, al_ref, ar_ref, dinv_ref,
                  atl_ref, atr_ref, hn_ref, aln_ref, arn_ref):
        agg = aggp_ref[0] + aggp_ref[1]
        dinv_n = dinv_ref[...]
        cself = jnp.tanh(al_ref[...] + ar_ref[...]) * dinv_n * dinv_n
        hn = agg + hprev_ref[...] * cself[:, None] + EPS * h0_ref[...]
        hn_ref[...] = hn
        aln_ref[...] = jnp.sum(hn * atl_ref[...][None, :], axis=1)
        arn_ref[...] = jnp.sum(hn * atr_ref[...][None, :], axis=1)

    def tc_combine(aggp, hprev, al, ar, att_l_next, att_r_next):
        return pl.pallas_call(
            tc_c_body,
            out_shape=[
                jax.ShapeDtypeStruct((NP, H), jnp.float32),
                jax.ShapeDtypeStruct((NP,), jnp.float32),
                jax.ShapeDtypeStruct((NP,), jnp.float32),
            ],
        )(aggp, hprev, h0, al, ar, dinv, att_l_next, att_r_next)

    aggp1 = edge_kernel(h0, al0, ar0, dinv, srcp, dstp)
    h1, al1, ar1 = tc_combine(aggp1, h0, al0, ar0, att_l1, att_r1)
    aggp2 = edge_kernel(h1, al1, ar1, dinv, srcp, dstp)

    def tc_e_body(aggp_ref, hprev_ref, h0_ref, al_ref, ar_ref, dinv_ref,
                  wout_ref, bout_ref, hout_ref, gsum_ref):
        agg = aggp_ref[0] + aggp_ref[1]
        dinv_n = dinv_ref[...]
        cself = jnp.tanh(al_ref[...] + ar_ref[...]) * dinv_n * dinv_n
        h2 = agg + hprev_ref[...] * cself[:, None] + EPS * h0_ref[...]
        rows = lax.broadcasted_iota(jnp.int32, (NP, H), 0)
        h2 = jnp.where(rows < N, h2, 0.0)
        wout = wout_ref[...]
        hout = (
            jnp.dot(h2, wout, preferred_element_type=jnp.float32)
            + bout_ref[...][None, :]
        )
        hout_ref[...] = hout
        colsum = jnp.sum(h2, axis=0)
        gsum_ref[...] = (
            jnp.dot(colsum, wout, preferred_element_type=jnp.float32)
            + float(N) * bout_ref[...]
        )[None, :]

    hout_p, gsum = pl.pallas_call(
        tc_e_body,
        out_shape=[
            jax.ShapeDtypeStruct((NP, H), jnp.float32),
            jax.ShapeDtypeStruct((1, H), jnp.float32),
        ],
    )(aggp2, h1, h0, al1, ar1, dinv, W_out, b_out)

    return gsum, hout_p[:N]


# speedup vs baseline: 47.2260x; 1.0836x over previous
import functools

import jax
import jax.numpy as jnp
from jax import lax
from jax.experimental import pallas as pl
from jax.experimental.pallas import tpu as pltpu
from jax.experimental.pallas import tpu_sc as plsc

EPS = 0.1
NC = 2
NS = 16
NW = NC * NS
CHUNK = 80
NSLOT = 3


def _sc_mesh():
    return plsc.VectorSubcoreMesh(
        core_axis_name="c", subcore_axis_name="s", num_cores=NC, num_subcores=NS
    )


def kernel(x, edge_index, W_in, b_in, att_l0, att_r0, att_l1, att_r1, W_out, b_out):
    N, D = x.shape
    H = W_in.shape[1]
    E = edge_index.shape[1]

    NP = -(-N // (NS * CHUNK)) * (NS * CHUNK)
    SUB = NP // NS
    n_chunks = -(--(-E // (NW * CHUNK)) // NSLOT) * NSLOT
    E_pad = n_chunks * NW * CHUNK

    src = edge_index[0]
    dst = edge_index[1]
    pad_n = E_pad - E
    pad_idx = (jnp.arange(pad_n, dtype=jnp.int32) % N).astype(jnp.int32)
    srcp = jnp.concatenate([src, pad_idx]).reshape(NW, n_chunks, CHUNK)
    dstp = jnp.concatenate([dst, pad_idx]).reshape(NW, n_chunks, CHUNK)
    edg = jnp.stack([srcp, dstp], axis=2)
    val = jnp.concatenate(
        [jnp.ones((E,), jnp.float32), jnp.zeros((pad_n,), jnp.float32)]
    ).reshape(NW, n_chunks, CHUNK)
    xp = jnp.pad(x, ((0, NP - N), (0, 0)))

    @functools.partial(
        pl.kernel,
        out_type=jax.ShapeDtypeStruct((NC, NP), jnp.float32),
        mesh=_sc_mesh(),
        compiler_params=pltpu.CompilerParams(needs_layout_passes=False),
        scratch_types=[
            pltpu.VMEM((n_chunks, CHUNK), jnp.int32),
            pltpu.VMEM((n_chunks, CHUNK), jnp.float32),
            pltpu.VMEM((SUB,), jnp.float32),
            pltpu.VMEM_SHARED((NP,), jnp.float32),
        ],
    )
    def deg_kernel(dst_hbm, val_hbm, out_hbm, dst_v, val_v, zbuf_v, hist_s):
        cc = lax.axis_index("c")
        ss = lax.axis_index("s")
        wid = ss * NC + cc
        pltpu.sync_copy(dst_hbm.at[wid], dst_v)
        pltpu.sync_copy(val_hbm.at[wid], val_v)

        def zb(i, _):
            zbuf_v[pl.ds(i * 16, 16)] = jnp.zeros((16,), jnp.float32)
            return 0

        lax.fori_loop(0, SUB // 16, zb, 0)
        pltpu.sync_copy(zbuf_v, hist_s.at[pl.ds(ss * SUB, SUB)])
        plsc.subcore_barrier()

        def body(ci, _):
            pltpu.sync_copy(val_v.at[ci], hist_s.at[dst_v.at[ci]], add=True)
            return 0

        lax.fori_loop(0, n_chunks, body, 0)
        plsc.subcore_barrier()
        pltpu.sync_copy(
            hist_s.at[pl.ds(ss * SUB, SUB)], out_hbm.at[cc, pl.ds(ss * SUB, SUB)]
        )

    degp = deg_kernel(dstp, val)

    def tc_a_body(x_ref, w_ref, b_ref, atl_ref, atr_ref, degp_ref,
                  h_ref, al_ref, ar_ref, dinv_ref):
        h = jnp.maximum(
            jnp.dot(x_ref[...], w_ref[...], preferred_element_type=jnp.float32)
            + b_ref[...][None, :],
            0.0,
        )
        h_ref[...] = h
        al_ref[...] = jnp.sum(h * atl_ref[...][None, :], axis=1)
        ar_ref[...] = jnp.sum(h * atr_ref[...][None, :], axis=1)
        deg = degp_ref[0, :] + degp_ref[1, :] + 1.0
        dinv_ref[...] = lax.rsqrt(deg)

    h0, al0, ar0, dinv = pl.pallas_call(
        tc_a_body,
        out_shape=[
            jax.ShapeDtypeStruct((NP, H), jnp.float32),
            jax.ShapeDtypeStruct((NP,), jnp.float32),
            jax.ShapeDtypeStruct((NP,), jnp.float32),
            jax.ShapeDtypeStruct((NP,), jnp.float32),
        ],
    )(xp, W_in, b_in, att_l0, att_r0, degp)

    @functools.partial(
        pl.kernel,
        out_type=jax.ShapeDtypeStruct((NC, NP, H), jnp.float32),
        mesh=_sc_mesh(),
        compiler_params=pltpu.CompilerParams(needs_layout_passes=False),
        scratch_types=[
            [pltpu.VMEM((2, CHUNK), jnp.int32)] * NSLOT,
            [pltpu.VMEM((CHUNK,), jnp.float32)] * NSLOT,
            [pltpu.VMEM((CHUNK,), jnp.float32)] * NSLOT,
            [pltpu.VMEM((CHUNK,), jnp.float32)] * NSLOT,
            [pltpu.VMEM((CHUNK,), jnp.float32)] * NSLOT,
            [pltpu.VMEM((CHUNK,), jnp.float32)] * NSLOT,
            [pltpu.VMEM((CHUNK, 128), jnp.float32)] * NSLOT,
            pltpu.VMEM_SHARED((NP,), jnp.float32),
            pltpu.VMEM_SHARED((NP,), jnp.float32),
            pltpu.VMEM_SHARED((NP,), jnp.float32),
            pltpu.VMEM_SHARED((NP, 128), jnp.float32),
            [pltpu.SemaphoreType.DMA] * NSLOT,
            [pltpu.SemaphoreType.DMA] * NSLOT,
            [pltpu.SemaphoreType.DMA] * NSLOT,
            [pltpu.SemaphoreType.DMA] * NSLOT,
        ],
    )
    def edge_kernel(h_hbm, al_hbm, ar_hbm, dinv_hbm, edg_hbm,
                    out_hbm, idxb, als, ard, dis, did, coef,
                    rows, al_sh, ar_sh, dinv_sh, agg_s,
                    sem_i, sem_r, sem_c, sem_s):
        srcb = [b.at[0] for b in idxb]
        dstb = [b.at[1] for b in idxb]
        cc = lax.axis_index("c")
        ss = lax.axis_index("s")
        wid = ss * NC + cc

        @pl.when(ss == 0)
        def _():
            pltpu.sync_copy(al_hbm, al_sh)

        @pl.when(ss == 1)
        def _():
            pltpu.sync_copy(ar_hbm, ar_sh)

        @pl.when(ss == 2)
        def _():
            pltpu.sync_copy(dinv_hbm, dinv_sh)

        def zrow(i, _):
            for j in range(8):
                rows[0][i, pl.ds(j * 16, 16)] = jnp.zeros((16,), jnp.float32)
            return 0

        lax.fori_loop(0, CHUNK, zrow, 0)
        for k in range(SUB // CHUNK):
            pltpu.sync_copy(rows[0], agg_s.at[pl.ds(ss * SUB + k * CHUNK, CHUNK)])
        plsc.subcore_barrier()

        def idx_dma(c, k):
            pltpu.async_copy(edg_hbm.at[wid, c], idxb[k], sem_i[k])

        def wait_idx(k):
            pltpu.make_async_copy(edg_hbm.at[wid, 0], idxb[k], sem_i[k]).wait()

        def issue_work(k):
            pltpu.async_copy(h_hbm.at[srcb[k]], rows[k], sem_r[k])
            pltpu.async_copy(al_sh.at[srcb[k]], als[k], sem_c[k])
            pltpu.async_copy(ar_sh.at[dstb[k]], ard[k], sem_c[k])
            pltpu.async_copy(dinv_sh.at[srcb[k]], dis[k], sem_c[k])
            pltpu.async_copy(dinv_sh.at[dstb[k]], did[k], sem_c[k])

        def wait_sct(k):
            pltpu.make_async_copy(rows[k], agg_s.at[dstb[k]], sem_s[k]).wait()

        def process(c, k):
            k1 = (k + 1) % NSLOT
            k2 = (k + 2) % NSLOT

            @pl.when(c + 1 < n_chunks)
            def _():
                wait_idx(k1)
                issue_work(k1)

            for _ in range(4):
                pltpu.make_async_copy(
                    al_sh.at[srcb[k]], als[k], sem_c[k]).wait()
            base_gid = (wid * n_chunks + c) * CHUNK
            for g in range(CHUNK // 16):
                av = als[k][pl.ds(g * 16, 16)]
                rv = ard[k][pl.ds(g * 16, 16)]
                dsv = dis[k][pl.ds(g * 16, 16)]
                ddv = did[k][pl.ds(g * 16, 16)]
                gid = base_gid + g * 16 + lax.iota(jnp.int32, 16)
                z = jnp.clip(av + rv, -15.0, 15.0)
                e = jnp.exp(2.0 * z)
                t = (e - 1.0) / (e + 1.0)
                coef[k][pl.ds(g * 16, 16)] = jnp.where(
                    gid < E, t * dsv * ddv, 0.0
                )

            pltpu.make_async_copy(h_hbm.at[srcb[k]], rows[k], sem_r[k]).wait()

            def srow(g, _):
                cvec = coef[k][pl.ds(g * 16, 16)]
                for r16 in range(16):
                    cs = cvec[r16]
                    row = g * 16 + r16
                    for j in range(8):
                        sl = rows[k][row, pl.ds(j * 16, 16)]
                        rows[k][row, pl.ds(j * 16, 16)] = sl * cs
                return 0

            lax.fori_loop(0, CHUNK // 16, srow, 0)
            pltpu.async_copy(rows[k], agg_s.at[dstb[k]], sem_s[k], add=True)

            @pl.when(c >= 1)
            def _():
                wait_sct(k2)

            @pl.when(c + 2 < n_chunks)
            def _():
                idx_dma(c + 2, k2)

        idx_dma(0, 0)
        idx_dma(1, 1)
        wait_idx(0)
        issue_work(0)

        def pipe_body(t, _):
            for k in range(NSLOT):
                process(NSLOT * t + k, k)
            return 0

        lax.fori_loop(0, n_chunks // NSLOT, pipe_body, 0)
        wait_sct((n_chunks - 1) % NSLOT)
        plsc.subcore_barrier()
        pltpu.sync_copy(
            agg_s.at[pl.ds(ss * SUB, SUB)], out_hbm.at[cc, pl.ds(ss * SUB, SUB)]
        )

    def tc_c_body(aggp_ref, hprev_ref, h0_ref, al_ref, ar_ref, dinv_ref,
                  atl_ref, atr_ref, hn_ref, aln_ref, arn_ref):
        agg = aggp_ref[0] + aggp_ref[1]
        dinv_n = dinv_ref[...]
        cself = jnp.tanh(al_ref[...] + ar_ref[...]) * dinv_n * dinv_n
        hn = agg + hprev_ref[...] * cself[:, None] + EPS * h0_ref[...]
        hn_ref[...] = hn
        aln_ref[...] = jnp.sum(hn * atl_ref[...][None, :], axis=1)
        arn_ref[...] = jnp.sum(hn * atr_ref[...][None, :], axis=1)

    def tc_combine(aggp, hprev, al, ar, att_l_next, att_r_next):
        return pl.pallas_call(
            tc_c_body,
            out_shape=[
                jax.ShapeDtypeStruct((NP, H), jnp.float32),
                jax.ShapeDtypeStruct((NP,), jnp.float32),
                jax.ShapeDtypeStruct((NP,), jnp.float32),
            ],
        )(aggp, hprev, h0, al, ar, dinv, att_l_next, att_r_next)

    aggp1 = edge_kernel(h0, al0, ar0, dinv, edg)
    h1, al1, ar1 = tc_combine(aggp1, h0, al0, ar0, att_l1, att_r1)
    aggp2 = edge_kernel(h1, al1, ar1, dinv, edg)

    def tc_e_body(aggp_ref, hprev_ref, h0_ref, al_ref, ar_ref, dinv_ref,
                  wout_ref, bout_ref, hout_ref, gsum_ref):
        agg = aggp_ref[0] + aggp_ref[1]
        dinv_n = dinv_ref[...]
        cself = jnp.tanh(al_ref[...] + ar_ref[...]) * dinv_n * dinv_n
        h2 = agg + hprev_ref[...] * cself[:, None] + EPS * h0_ref[...]
        rows = lax.broadcasted_iota(jnp.int32, (NP, H), 0)
        h2 = jnp.where(rows < N, h2, 0.0)
        wout = wout_ref[...]
        hout = (
            jnp.dot(h2, wout, preferred_element_type=jnp.float32)
            + bout_ref[...][None, :]
        )
        hout_ref[...] = hout
        colsum = jnp.sum(h2, axis=0)
        gsum_ref[...] = (
            jnp.dot(colsum, wout, preferred_element_type=jnp.float32)
            + float(N) * bout_ref[...]
        )[None, :]

    hout_p, gsum = pl.pallas_call(
        tc_e_body,
        out_shape=[
            jax.ShapeDtypeStruct((NP, H), jnp.float32),
            jax.ShapeDtypeStruct((1, H), jnp.float32),
        ],
    )(aggp2, h1, h0, al1, ar1, dinv, W_out, b_out)

    return gsum, hout_p[:N]


# speedup vs baseline: 47.4495x; 1.0047x over previous
import functools

import jax
import jax.numpy as jnp
from jax import lax
from jax.experimental import pallas as pl
from jax.experimental.pallas import tpu as pltpu
from jax.experimental.pallas import tpu_sc as plsc

EPS = 0.1
NC = 2
NS = 16
NW = NC * NS
CHUNK = 80
NSLOT = 3


def _sc_mesh():
    return plsc.VectorSubcoreMesh(
        core_axis_name="c", subcore_axis_name="s", num_cores=NC, num_subcores=NS
    )


def kernel(x, edge_index, W_in, b_in, att_l0, att_r0, att_l1, att_r1, W_out, b_out):
    N, D = x.shape
    H = W_in.shape[1]
    E = edge_index.shape[1]

    NP = -(-N // (NS * CHUNK)) * (NS * CHUNK)
    SUB = NP // NS
    n_chunks = -(--(-E // (NW * CHUNK)) // NSLOT) * NSLOT
    E_pad = n_chunks * NW * CHUNK

    src = edge_index[0]
    dst = edge_index[1]
    pad_n = E_pad - E
    pad_idx = (jnp.arange(pad_n, dtype=jnp.int32) % N).astype(jnp.int32)
    srcp = jnp.concatenate([src, pad_idx]).reshape(NW, n_chunks, CHUNK)
    dstp = jnp.concatenate([dst, pad_idx]).reshape(NW, n_chunks, CHUNK)
    edg = jnp.stack([srcp, dstp], axis=2)
    xp = jnp.pad(x, ((0, NP - N), (0, 0)))

    @functools.partial(
        pl.kernel,
        out_type=jax.ShapeDtypeStruct((NC, NP), jnp.float32),
        mesh=_sc_mesh(),
        compiler_params=pltpu.CompilerParams(needs_layout_passes=False),
        scratch_types=[
            pltpu.VMEM((n_chunks, CHUNK), jnp.int32),
            pltpu.VMEM((n_chunks, CHUNK), jnp.float32),
            pltpu.VMEM((SUB,), jnp.float32),
            pltpu.VMEM_SHARED((NP,), jnp.float32),
            pltpu.SemaphoreType.DMA,
        ],
    )
    def deg_kernel(dst_hbm, out_hbm, dst_v, val_v, zbuf_v, hist_s, dsem):
        cc = lax.axis_index("c")
        ss = lax.axis_index("s")
        wid = ss * NC + cc
        pltpu.sync_copy(dst_hbm.at[wid], dst_v)

        def zb(i, _):
            zbuf_v[pl.ds(i * 16, 16)] = jnp.zeros((16,), jnp.float32)
            return 0

        lax.fori_loop(0, SUB // 16, zb, 0)
        pltpu.sync_copy(zbuf_v, hist_s.at[pl.ds(ss * SUB, SUB)])

        base_gid = wid * n_chunks * CHUNK

        def vb(i, _):
            for g in range(CHUNK // 16):
                gid = base_gid + i * CHUNK + g * 16 + lax.iota(jnp.int32, 16)
                val_v[i, pl.ds(g * 16, 16)] = jnp.where(gid < E, 1.0, 0.0)
            return 0

        lax.fori_loop(0, n_chunks, vb, 0)
        plsc.subcore_barrier()

        def body(ci, _):
            pltpu.async_copy(val_v.at[ci], hist_s.at[dst_v.at[ci]], dsem,
                             add=True)
            return 0

        lax.fori_loop(0, n_chunks, body, 0)

        def drain(ci, _):
            pltpu.make_async_copy(
                val_v.at[0], hist_s.at[dst_v.at[0]], dsem).wait()
            return 0

        lax.fori_loop(0, n_chunks, drain, 0)
        plsc.subcore_barrier()
        pltpu.sync_copy(
            hist_s.at[pl.ds(ss * SUB, SUB)], out_hbm.at[cc, pl.ds(ss * SUB, SUB)]
        )

    degp = deg_kernel(dstp)

    def tc_a_body(x_ref, w_ref, b_ref, atl_ref, atr_ref, degp_ref,
                  h_ref, al_ref, ar_ref, dinv_ref):
        h = jnp.maximum(
            jnp.dot(x_ref[...], w_ref[...], preferred_element_type=jnp.float32)
            + b_ref[...][None, :],
            0.0,
        )
        h_ref[...] = h
        al_ref[...] = jnp.sum(h * atl_ref[...][None, :], axis=1)
        ar_ref[...] = jnp.sum(h * atr_ref[...][None, :], axis=1)
        deg = degp_ref[0, :] + degp_ref[1, :] + 1.0
        dinv_ref[...] = lax.rsqrt(deg)

    h0, al0, ar0, dinv = pl.pallas_call(
        tc_a_body,
        out_shape=[
            jax.ShapeDtypeStruct((NP, H), jnp.float32),
            jax.ShapeDtypeStruct((NP,), jnp.float32),
            jax.ShapeDtypeStruct((NP,), jnp.float32),
            jax.ShapeDtypeStruct((NP,), jnp.float32),
        ],
    )(xp, W_in, b_in, att_l0, att_r0, degp)

    @functools.partial(
        pl.kernel,
        out_type=jax.ShapeDtypeStruct((NC, NP, H), jnp.float32),
        mesh=_sc_mesh(),
        compiler_params=pltpu.CompilerParams(needs_layout_passes=False),
        scratch_types=[
            [pltpu.VMEM((2, CHUNK), jnp.int32)] * NSLOT,
            [pltpu.VMEM((4 * CHUNK,), jnp.float32)] * NSLOT,
            [pltpu.VMEM((CHUNK,), jnp.float32)] * NSLOT,
            [pltpu.VMEM((CHUNK, 128), jnp.float32)] * NSLOT,
            pltpu.VMEM_SHARED((NP,), jnp.float32),
            pltpu.VMEM_SHARED((NP,), jnp.float32),
            pltpu.VMEM_SHARED((NP,), jnp.float32),
            pltpu.VMEM_SHARED((NP, 128), jnp.float32),
            [pltpu.SemaphoreType.DMA] * NSLOT,
            [pltpu.SemaphoreType.DMA] * NSLOT,
            [pltpu.SemaphoreType.DMA] * NSLOT,
            [pltpu.SemaphoreType.DMA] * NSLOT,
        ],
    )
    def edge_kernel(h_hbm, al_hbm, ar_hbm, dinv_hbm, edg_hbm,
                    out_hbm, idxb, cbuf, coef,
                    rows, al_sh, ar_sh, dinv_sh, agg_s,
                    sem_i, sem_r, sem_c, sem_s):
        als = [b.at[pl.ds(0 * CHUNK, CHUNK)] for b in cbuf]
        ard = [b.at[pl.ds(1 * CHUNK, CHUNK)] for b in cbuf]
        dis = [b.at[pl.ds(2 * CHUNK, CHUNK)] for b in cbuf]
        did = [b.at[pl.ds(3 * CHUNK, CHUNK)] for b in cbuf]
        srcb = [b.at[0] for b in idxb]
        dstb = [b.at[1] for b in idxb]
        cc = lax.axis_index("c")
        ss = lax.axis_index("s")
        wid = ss * NC + cc

        @pl.when(ss == 0)
        def _():
            pltpu.sync_copy(al_hbm, al_sh)

        @pl.when(ss == 1)
        def _():
            pltpu.sync_copy(ar_hbm, ar_sh)

        @pl.when(ss == 2)
        def _():
            pltpu.sync_copy(dinv_hbm, dinv_sh)

        def zrow(i, _):
            for j in range(8):
                rows[0][i, pl.ds(j * 16, 16)] = jnp.zeros((16,), jnp.float32)
            return 0

        lax.fori_loop(0, CHUNK, zrow, 0)
        for k in range(SUB // CHUNK):
            pltpu.sync_copy(rows[0], agg_s.at[pl.ds(ss * SUB + k * CHUNK, CHUNK)])
        plsc.subcore_barrier()

        def idx_dma(c, k):
            pltpu.async_copy(edg_hbm.at[wid, c], idxb[k], sem_i[k])

        def wait_idx(k):
            pltpu.make_async_copy(edg_hbm.at[wid, 0], idxb[k], sem_i[k]).wait()

        def issue_work(k):
            pltpu.async_copy(h_hbm.at[srcb[k]], rows[k], sem_r[k])
            pltpu.async_copy(al_sh.at[srcb[k]], als[k], sem_c[k])
            pltpu.async_copy(ar_sh.at[dstb[k]], ard[k], sem_c[k])
            pltpu.async_copy(dinv_sh.at[srcb[k]], dis[k], sem_c[k])
            pltpu.async_copy(dinv_sh.at[dstb[k]], did[k], sem_c[k])

        def wait_sct(k):
            pltpu.make_async_copy(rows[k], agg_s.at[dstb[k]], sem_s[k]).wait()

        def process(c, k):
            k1 = (k + 1) % NSLOT
            k2 = (k + 2) % NSLOT

            @pl.when(c + 1 < n_chunks)
            def _():
                wait_idx(k1)
                issue_work(k1)

            pltpu.make_async_copy(
                al_sh.at[pl.ds(0, 4 * CHUNK)], cbuf[k], sem_c[k]).wait()
            base_gid = (wid * n_chunks + c) * CHUNK
            for g in range(CHUNK // 16):
                av = als[k][pl.ds(g * 16, 16)]
                rv = ard[k][pl.ds(g * 16, 16)]
                dsv = dis[k][pl.ds(g * 16, 16)]
                ddv = did[k][pl.ds(g * 16, 16)]
                gid = base_gid + g * 16 + lax.iota(jnp.int32, 16)
                z = jnp.clip(av + rv, -15.0, 15.0)
                e = jnp.exp(2.0 * z)
                t = (e - 1.0) / (e + 1.0)
                coef[k][pl.ds(g * 16, 16)] = jnp.where(
                    gid < E, t * dsv * ddv, 0.0
                )

            pltpu.make_async_copy(h_hbm.at[srcb[k]], rows[k], sem_r[k]).wait()

            def srow(g, _):
                cvec = coef[k][pl.ds(g * 16, 16)]
                for r16 in range(16):
                    cs = cvec[r16]
                    row = g * 16 + r16
                    for j in range(8):
                        sl = rows[k][row, pl.ds(j * 16, 16)]
                        rows[k][row, pl.ds(j * 16, 16)] = sl * cs
                return 0

            lax.fori_loop(0, CHUNK // 16, srow, 0)
            pltpu.async_copy(rows[k], agg_s.at[dstb[k]], sem_s[k], add=True)

            @pl.when(c >= 1)
            def _():
                wait_sct(k2)

            @pl.when(c + 2 < n_chunks)
            def _():
                idx_dma(c + 2, k2)

        idx_dma(0, 0)
        idx_dma(1, 1)
        wait_idx(0)
        issue_work(0)

        def pipe_body(t, _):
            for k in range(NSLOT):
                process(NSLOT * t + k, k)
            return 0

        lax.fori_loop(0, n_chunks // NSLOT, pipe_body, 0)
        wait_sct((n_chunks - 1) % NSLOT)
        plsc.subcore_barrier()
        pltpu.sync_copy(
            agg_s.at[pl.ds(ss * SUB, SUB)], out_hbm.at[cc, pl.ds(ss * SUB, SUB)]
        )

    def tc_c_body(aggp_ref, hprev_ref, h0_ref, al_ref, ar_ref, dinv_ref,
                  atl_ref, atr_ref, hn_ref, aln_ref, arn_ref):
        agg = aggp_ref[0] + aggp_ref[1]
        dinv_n = dinv_ref[...]
        cself = jnp.tanh(al_ref[...] + ar_ref[...]) * dinv_n * dinv_n
        hn = agg + hprev_ref[...] * cself[:, None] + EPS * h0_ref[...]
        hn_ref[...] = hn
        aln_ref[...] = jnp.sum(hn * atl_ref[...][None, :], axis=1)
        arn_ref[...] = jnp.sum(hn * atr_ref[...][None, :], axis=1)

    def tc_combine(aggp, hprev, al, ar, att_l_next, att_r_next):
        return pl.pallas_call(
            tc_c_body,
            out_shape=[
                jax.ShapeDtypeStruct((NP, H), jnp.float32),
                jax.ShapeDtypeStruct((NP,), jnp.float32),
                jax.ShapeDtypeStruct((NP,), jnp.float32),
            ],
        )(aggp, hprev, h0, al, ar, dinv, att_l_next, att_r_next)

    aggp1 = edge_kernel(h0, al0, ar0, dinv, edg)
    h1, al1, ar1 = tc_combine(aggp1, h0, al0, ar0, att_l1, att_r1)
    aggp2 = edge_kernel(h1, al1, ar1, dinv, edg)

    def tc_e_body(aggp_ref, hprev_ref, h0_ref, al_ref, ar_ref, dinv_ref,
                  wout_ref, bout_ref, hout_ref, gsum_ref):
        agg = aggp_ref[0] + aggp_ref[1]
        dinv_n = dinv_ref[...]
        cself = jnp.tanh(al_ref[...] + ar_ref[...]) * dinv_n * dinv_n
        h2 = agg + hprev_ref[...] * cself[:, None] + EPS * h0_ref[...]
        rows = lax.broadcasted_iota(jnp.int32, (NP, H), 0)
        h2 = jnp.where(rows < N, h2, 0.0)
        wout = wout_ref[...]
        hout = (
            jnp.dot(h2, wout, preferred_element_type=jnp.float32)
            + bout_ref[...][None, :]
        )
        hout_ref[...] = hout
        colsum = jnp.sum(h2, axis=0)
        gsum_ref[...] = (
            jnp.dot(colsum, wout, preferred_element_type=jnp.float32)
            + float(N) * bout_ref[...]
        )[None, :]

    hout_p, gsum = pl.pallas_call(
        tc_e_body,
        out_shape=[
            jax.ShapeDtypeStruct((NP, H), jnp.float32),
            jax.ShapeDtypeStruct((1, H), jnp.float32),
        ],
    )(aggp2, h1, h0, al1, ar1, dinv, W_out, b_out)

    return gsum, hout_p[:N]


# speedup vs baseline: 47.8574x; 1.0086x over previous
import functools

import jax
import jax.numpy as jnp
from jax import lax
from jax.experimental import pallas as pl
from jax.experimental.pallas import tpu as pltpu
from jax.experimental.pallas import tpu_sc as plsc

EPS = 0.1
NC = 2
NS = 16
NW = NC * NS
CHUNK = 80
NSLOT = 3


def _sc_mesh():
    return plsc.VectorSubcoreMesh(
        core_axis_name="c", subcore_axis_name="s", num_cores=NC, num_subcores=NS
    )


def kernel(x, edge_index, W_in, b_in, att_l0, att_r0, att_l1, att_r1, W_out, b_out):
    N, D = x.shape
    H = W_in.shape[1]
    E = edge_index.shape[1]

    NP = -(-N // (NS * CHUNK)) * (NS * CHUNK)
    SUB = NP // NS
    n_chunks = -(--(-E // (NW * CHUNK)) // NSLOT) * NSLOT
    E_pad = n_chunks * NW * CHUNK

    src = edge_index[0]
    dst = edge_index[1]
    pad_n = E_pad - E
    pad_idx = (jnp.arange(pad_n, dtype=jnp.int32) % N).astype(jnp.int32)
    srcp = jnp.concatenate([src, pad_idx]).reshape(NW, n_chunks, CHUNK)
    dstp = jnp.concatenate([dst, pad_idx]).reshape(NW, n_chunks, CHUNK)
    edg = jnp.stack([srcp, dstp], axis=2)

    @functools.partial(
        pl.kernel,
        out_type=jax.ShapeDtypeStruct((NC, NP), jnp.float32),
        mesh=_sc_mesh(),
        compiler_params=pltpu.CompilerParams(needs_layout_passes=False),
        scratch_types=[
            pltpu.VMEM((n_chunks, CHUNK), jnp.int32),
            pltpu.VMEM((n_chunks, CHUNK), jnp.float32),
            pltpu.VMEM((SUB,), jnp.float32),
            pltpu.VMEM_SHARED((NP,), jnp.float32),
            pltpu.SemaphoreType.DMA,
        ],
    )
    def deg_kernel(dst_hbm, out_hbm, dst_v, val_v, zbuf_v, hist_s, dsem):
        cc = lax.axis_index("c")
        ss = lax.axis_index("s")
        wid = ss * NC + cc
        pltpu.sync_copy(dst_hbm.at[wid], dst_v)

        def zb(i, _):
            zbuf_v[pl.ds(i * 16, 16)] = jnp.zeros((16,), jnp.float32)
            return 0

        lax.fori_loop(0, SUB // 16, zb, 0)
        pltpu.sync_copy(zbuf_v, hist_s.at[pl.ds(ss * SUB, SUB)])

        base_gid = wid * n_chunks * CHUNK

        def vb(i, _):
            for g in range(CHUNK // 16):
                gid = base_gid + i * CHUNK + g * 16 + lax.iota(jnp.int32, 16)
                val_v[i, pl.ds(g * 16, 16)] = jnp.where(gid < E, 1.0, 0.0)
            return 0

        lax.fori_loop(0, n_chunks, vb, 0)
        plsc.subcore_barrier()

        def body(ci, _):
            pltpu.async_copy(val_v.at[ci], hist_s.at[dst_v.at[ci]], dsem,
                             add=True)
            return 0

        lax.fori_loop(0, n_chunks, body, 0)

        def drain(ci, _):
            pltpu.make_async_copy(
                val_v.at[0], hist_s.at[dst_v.at[0]], dsem).wait()
            return 0

        lax.fori_loop(0, n_chunks, drain, 0)
        plsc.subcore_barrier()
        pltpu.sync_copy(
            hist_s.at[pl.ds(ss * SUB, SUB)], out_hbm.at[cc, pl.ds(ss * SUB, SUB)]
        )

    degp = deg_kernel(dstp)

    def tc_a_body(x_ref, w_ref, b_ref, atl_ref, atr_ref, degp_ref,
                  h_ref, al_ref, ar_ref, dinv_ref):
        h = jnp.maximum(
            jnp.dot(x_ref[...], w_ref[...], preferred_element_type=jnp.float32)
            + b_ref[...][None, :],
            0.0,
        )
        h = jnp.concatenate(
            [h, jnp.zeros((NP - N, h.shape[1]), jnp.float32)], axis=0
        )
        h_ref[...] = h
        al_ref[...] = jnp.sum(h * atl_ref[...][None, :], axis=1)
        ar_ref[...] = jnp.sum(h * atr_ref[...][None, :], axis=1)
        deg = degp_ref[0, :] + degp_ref[1, :] + 1.0
        dinv_ref[...] = lax.rsqrt(deg)

    h0, al0, ar0, dinv = pl.pallas_call(
        tc_a_body,
        out_shape=[
            jax.ShapeDtypeStruct((NP, H), jnp.float32),
            jax.ShapeDtypeStruct((NP,), jnp.float32),
            jax.ShapeDtypeStruct((NP,), jnp.float32),
            jax.ShapeDtypeStruct((NP,), jnp.float32),
        ],
    )(x, W_in, b_in, att_l0, att_r0, degp)

    @functools.partial(
        pl.kernel,
        out_type=jax.ShapeDtypeStruct((NC, NP, H), jnp.float32),
        mesh=_sc_mesh(),
        compiler_params=pltpu.CompilerParams(needs_layout_passes=False),
        scratch_types=[
            [pltpu.VMEM((2, CHUNK), jnp.int32)] * NSLOT,
            [pltpu.VMEM((4 * CHUNK,), jnp.float32)] * NSLOT,
            [pltpu.VMEM((CHUNK,), jnp.float32)] * NSLOT,
            [pltpu.VMEM((CHUNK, 128), jnp.float32)] * NSLOT,
            pltpu.VMEM_SHARED((NP,), jnp.float32),
            pltpu.VMEM_SHARED((NP,), jnp.float32),
            pltpu.VMEM_SHARED((NP,), jnp.float32),
            pltpu.VMEM_SHARED((NP, 128), jnp.float32),
            [pltpu.SemaphoreType.DMA] * NSLOT,
            [pltpu.SemaphoreType.DMA] * NSLOT,
            [pltpu.SemaphoreType.DMA] * NSLOT,
            [pltpu.SemaphoreType.DMA] * NSLOT,
        ],
    )
    def edge_kernel(h_hbm, al_hbm, ar_hbm, dinv_hbm, edg_hbm,
                    out_hbm, idxb, cbuf, coef,
                    rows, al_sh, ar_sh, dinv_sh, agg_s,
                    sem_i, sem_r, sem_c, sem_s):
        als = [b.at[pl.ds(0 * CHUNK, CHUNK)] for b in cbuf]
        ard = [b.at[pl.ds(1 * CHUNK, CHUNK)] for b in cbuf]
        dis = [b.at[pl.ds(2 * CHUNK, CHUNK)] for b in cbuf]
        did = [b.at[pl.ds(3 * CHUNK, CHUNK)] for b in cbuf]
        srcb = [b.at[0] for b in idxb]
        dstb = [b.at[1] for b in idxb]
        cc = lax.axis_index("c")
        ss = lax.axis_index("s")
        wid = ss * NC + cc

        @pl.when(ss == 0)
        def _():
            pltpu.sync_copy(al_hbm, al_sh)

        @pl.when(ss == 1)
        def _():
            pltpu.sync_copy(ar_hbm, ar_sh)

        @pl.when(ss == 2)
        def _():
            pltpu.sync_copy(dinv_hbm, dinv_sh)

        def zrow(i, _):
            for j in range(8):
                rows[0][i, pl.ds(j * 16, 16)] = jnp.zeros((16,), jnp.float32)
            return 0

        lax.fori_loop(0, CHUNK, zrow, 0)
        for k in range(SUB // CHUNK):
            pltpu.sync_copy(rows[0], agg_s.at[pl.ds(ss * SUB + k * CHUNK, CHUNK)])
        plsc.subcore_barrier()

        def idx_dma(c, k):
            pltpu.async_copy(edg_hbm.at[wid, c], idxb[k], sem_i[k])

        def wait_idx(k):
            pltpu.make_async_copy(edg_hbm.at[wid, 0], idxb[k], sem_i[k]).wait()

        def issue_work(k):
            pltpu.async_copy(h_hbm.at[srcb[k]], rows[k], sem_r[k])
            pltpu.async_copy(al_sh.at[srcb[k]], als[k], sem_c[k])
            pltpu.async_copy(ar_sh.at[dstb[k]], ard[k], sem_c[k])
            pltpu.async_copy(dinv_sh.at[srcb[k]], dis[k], sem_c[k])
            pltpu.async_copy(dinv_sh.at[dstb[k]], did[k], sem_c[k])

        def wait_sct(k):
            pltpu.make_async_copy(rows[k], agg_s.at[dstb[k]], sem_s[k]).wait()

        def process(c, k):
            k1 = (k + 1) % NSLOT
            k2 = (k + 2) % NSLOT

            @pl.when(c + 1 < n_chunks)
            def _():
                wait_idx(k1)
                issue_work(k1)

            pltpu.make_async_copy(
                al_sh.at[pl.ds(0, 4 * CHUNK)], cbuf[k], sem_c[k]).wait()
            base_gid = (wid * n_chunks + c) * CHUNK
            for g in range(CHUNK // 16):
                av = als[k][pl.ds(g * 16, 16)]
                rv = ard[k][pl.ds(g * 16, 16)]
                dsv = dis[k][pl.ds(g * 16, 16)]
                ddv = did[k][pl.ds(g * 16, 16)]
                gid = base_gid + g * 16 + lax.iota(jnp.int32, 16)
                z = jnp.clip(av + rv, -15.0, 15.0)
                e = jnp.exp(2.0 * z)
                t = (e - 1.0) / (e + 1.0)
                coef[k][pl.ds(g * 16, 16)] = jnp.where(
                    gid < E, t * dsv * ddv, 0.0
                )

            pltpu.make_async_copy(h_hbm.at[srcb[k]], rows[k], sem_r[k]).wait()

            def srow(g, _):
                cvec = coef[k][pl.ds(g * 16, 16)]
                for r16 in range(16):
                    cs = cvec[r16]
                    row = g * 16 + r16
                    for j in range(8):
                        sl = rows[k][row, pl.ds(j * 16, 16)]
                        rows[k][row, pl.ds(j * 16, 16)] = sl * cs
                return 0

            lax.fori_loop(0, CHUNK // 16, srow, 0)
            pltpu.async_copy(rows[k], agg_s.at[dstb[k]], sem_s[k], add=True)

            @pl.when(c >= 1)
            def _():
                wait_sct(k2)

            @pl.when(c + 2 < n_chunks)
            def _():
                idx_dma(c + 2, k2)

        idx_dma(0, 0)
        idx_dma(1, 1)
        wait_idx(0)
        issue_work(0)

        def pipe_body(t, _):
            for k in range(NSLOT):
                process(NSLOT * t + k, k)
            return 0

        lax.fori_loop(0, n_chunks // NSLOT, pipe_body, 0)
        wait_sct((n_chunks - 1) % NSLOT)
        plsc.subcore_barrier()
        pltpu.sync_copy(
            agg_s.at[pl.ds(ss * SUB, SUB)], out_hbm.at[cc, pl.ds(ss * SUB, SUB)]
        )

    def tc_c_body(aggp_ref, hprev_ref, h0_ref, al_ref, ar_ref, dinv_ref,
                  atl_ref, atr_ref, hn_ref, aln_ref, arn_ref):
        agg = aggp_ref[0] + aggp_ref[1]
        dinv_n = dinv_ref[...]
        cself = jnp.tanh(al_ref[...] + ar_ref[...]) * dinv_n * dinv_n
        hn = agg + hprev_ref[...] * cself[:, None] + EPS * h0_ref[...]
        hn_ref[...] = hn
        aln_ref[...] = jnp.sum(hn * atl_ref[...][None, :], axis=1)
        arn_ref[...] = jnp.sum(hn * atr_ref[...][None, :], axis=1)

    def tc_combine(aggp, hprev, al, ar, att_l_next, att_r_next):
        return pl.pallas_call(
            tc_c_body,
            out_shape=[
                jax.ShapeDtypeStruct((NP, H), jnp.float32),
                jax.ShapeDtypeStruct((NP,), jnp.float32),
                jax.ShapeDtypeStruct((NP,), jnp.float32),
            ],
        )(aggp, hprev, h0, al, ar, dinv, att_l_next, att_r_next)

    aggp1 = edge_kernel(h0, al0, ar0, dinv, edg)
    h1, al1, ar1 = tc_combine(aggp1, h0, al0, ar0, att_l1, att_r1)
    aggp2 = edge_kernel(h1, al1, ar1, dinv, edg)

    def tc_e_body(aggp_ref, hprev_ref, h0_ref, al_ref, ar_ref, dinv_ref,
                  wout_ref, bout_ref, hout_ref, gsum_ref):
        agg = aggp_ref[0] + aggp_ref[1]
        dinv_n = dinv_ref[...]
        cself = jnp.tanh(al_ref[...] + ar_ref[...]) * dinv_n * dinv_n
        h2 = agg + hprev_ref[...] * cself[:, None] + EPS * h0_ref[...]
        rows = lax.broadcasted_iota(jnp.int32, (NP, H), 0)
        h2 = jnp.where(rows < N, h2, 0.0)
        wout = wout_ref[...]
        hout = (
            jnp.dot(h2, wout, preferred_element_type=jnp.float32)
            + bout_ref[...][None, :]
        )
        hout_ref[...] = hout
        colsum = jnp.sum(h2, axis=0)
        gsum_ref[...] = (
            jnp.dot(colsum, wout, preferred_element_type=jnp.float32)
            + float(N) * bout_ref[...]
        )[None, :]

    hout_p, gsum = pl.pallas_call(
        tc_e_body,
        out_shape=[
            jax.ShapeDtypeStruct((NP, H), jnp.float32),
            jax.ShapeDtypeStruct((1, H), jnp.float32),
        ],
    )(aggp2, h1, h0, al1, ar1, dinv, W_out, b_out)

    return gsum, hout_p[:N]
